# baseline (speedup 1.0000x reference)
import functools

import jax
import jax.numpy as jnp
from jax.experimental import pallas as pl


def _moe_tile_kernel(x_ref, wg_ref, bg_ref, w1_ref, b1_ref, w2_ref, b2_ref,
                     o_ref, *, n_experts):
    x = x_ref[...]
    logits = jnp.dot(x, wg_ref[...], preferred_element_type=jnp.float32)
    logits = logits + bg_ref[...]
    v1 = jnp.max(logits, axis=-1, keepdims=True)
    idx = jax.lax.broadcasted_iota(jnp.int32, logits.shape, 1)
    am = jnp.min(jnp.where(logits == v1, idx, n_experts), axis=-1, keepdims=True)
    neg = jnp.float32(-jnp.inf)
    v2 = jnp.max(jnp.where(idx == am, neg, logits), axis=-1, keepdims=True)
    scale = 2.0 - jax.nn.sigmoid(v1 - v2)

    xb = x.astype(jnp.bfloat16)
    u = jnp.dot(xb, w1_ref[...], preferred_element_type=jnp.float32)
    u = jnp.maximum(u + b1_ref[...], 0.0).astype(jnp.bfloat16)
    h = jnp.dot(u, w2_ref[...], preferred_element_type=jnp.float32)
    h = h + b2_ref[...]
    o_ref[...] = scale * h


def kernel(x, Wg, bg, W1, b1, W2, b2):
    B, N, D = x.shape
    T = B * N
    E = Wg.shape[1]
    H = W1.shape[1]
    xf = x.reshape(T, D)
    TILE = 512
    out = pl.pallas_call(
        functools.partial(_moe_tile_kernel, n_experts=E),
        grid=(T // TILE,),
        in_specs=[
            pl.BlockSpec((TILE, D), lambda i: (i, 0)),
            pl.BlockSpec((D, E), lambda i: (0, 0)),
            pl.BlockSpec((1, E), lambda i: (0, 0)),
            pl.BlockSpec((D, H), lambda i: (0, 0)),
            pl.BlockSpec((1, H), lambda i: (0, 0)),
            pl.BlockSpec((H, D), lambda i: (0, 0)),
            pl.BlockSpec((1, D), lambda i: (0, 0)),
        ],
        out_specs=pl.BlockSpec((TILE, D), lambda i: (i, 0)),
        out_shape=jax.ShapeDtypeStruct((T, D), jnp.float32),
    )(xf, Wg, bg.reshape(1, E), W1.astype(jnp.bfloat16), b1.reshape(1, H),
      W2.astype(jnp.bfloat16), b2.reshape(1, D))
    return out.reshape(B, N, D)


# speedup vs baseline: 1.1179x; 1.1179x over previous
import functools

import jax
import jax.numpy as jnp
from jax.experimental import pallas as pl


def _moe_tile_kernel(x_ref, wg_ref, bg_ref, w1_ref, b1_ref, w2_ref, b2_ref,
                     o_ref, *, n_experts):
    x = x_ref[...]
    logits = jnp.dot(x, wg_ref[...], preferred_element_type=jnp.float32)
    logits = logits + bg_ref[...]
    v1 = jnp.max(logits, axis=-1, keepdims=True)
    idx = jax.lax.broadcasted_iota(jnp.int32, logits.shape, 1)
    am = jnp.min(jnp.where(logits == v1, idx, n_experts), axis=-1, keepdims=True)
    neg = jnp.float32(-jnp.inf)
    v2 = jnp.max(jnp.where(idx == am, neg, logits), axis=-1, keepdims=True)
    scale = 2.0 - jax.nn.sigmoid(v1 - v2)

    u = jnp.dot(x, w1_ref[...], preferred_element_type=jnp.float32)
    u = jnp.maximum(u + b1_ref[...], 0.0)
    h = jnp.dot(u, w2_ref[...], preferred_element_type=jnp.float32)
    h = h + b2_ref[...]
    o_ref[...] = scale * h


def kernel(x, Wg, bg, W1, b1, W2, b2):
    B, N, D = x.shape
    T = B * N
    E = Wg.shape[1]
    H = W1.shape[1]
    xf = x.reshape(T, D)
    TILE = 256
    out = pl.pallas_call(
        functools.partial(_moe_tile_kernel, n_experts=E),
        grid=(T // TILE,),
        in_specs=[
            pl.BlockSpec((TILE, D), lambda i: (i, 0)),
            pl.BlockSpec((D, E), lambda i: (0, 0)),
            pl.BlockSpec((1, E), lambda i: (0, 0)),
            pl.BlockSpec((D, H), lambda i: (0, 0)),
            pl.BlockSpec((1, H), lambda i: (0, 0)),
            pl.BlockSpec((H, D), lambda i: (0, 0)),
            pl.BlockSpec((1, D), lambda i: (0, 0)),
        ],
        out_specs=pl.BlockSpec((TILE, D), lambda i: (i, 0)),
        out_shape=jax.ShapeDtypeStruct((T, D), jnp.float32),
    )(xf, Wg, bg.reshape(1, E), W1, b1.reshape(1, H), W2, b2.reshape(1, D))
    return out.reshape(B, N, D)


# speedup vs baseline: 1.1463x; 1.0254x over previous
import functools

import jax
import jax.numpy as jnp
from jax.experimental import pallas as pl


def _moe_tile_kernel(x_ref, wg_ref, bg_ref, w1_ref, b1_ref, w2_ref, b2_ref,
                     o_ref, *, n_experts):
    x = x_ref[...]
    logits = jnp.dot(x, wg_ref[...], preferred_element_type=jnp.float32)
    logits = logits + bg_ref[...]
    v1 = jnp.max(logits, axis=-1, keepdims=True)
    idx = jax.lax.broadcasted_iota(jnp.int32, logits.shape, 1)
    am = jnp.min(jnp.where(logits == v1, idx, n_experts), axis=-1, keepdims=True)
    neg = jnp.float32(-jnp.inf)
    v2 = jnp.max(jnp.where(idx == am, neg, logits), axis=-1, keepdims=True)
    scale = 2.0 - jax.nn.sigmoid(v1 - v2)

    u = jnp.dot(x, w1_ref[...], preferred_element_type=jnp.float32)
    u = jnp.maximum(u + b1_ref[...], 0.0)
    h = jnp.dot(u, w2_ref[...], preferred_element_type=jnp.float32)
    h = h + b2_ref[...]
    o_ref[...] = scale * h


def kernel(x, Wg, bg, W1, b1, W2, b2):
    B, N, D = x.shape
    T = B * N
    E = Wg.shape[1]
    H = W1.shape[1]
    xf = x.reshape(T, D)
    TILE = 512
    out = pl.pallas_call(
        functools.partial(_moe_tile_kernel, n_experts=E),
        grid=(T // TILE,),
        in_specs=[
            pl.BlockSpec((TILE, D), lambda i: (i, 0)),
            pl.BlockSpec((D, E), lambda i: (0, 0)),
            pl.BlockSpec((1, E), lambda i: (0, 0)),
            pl.BlockSpec((D, H), lambda i: (0, 0)),
            pl.BlockSpec((1, H), lambda i: (0, 0)),
            pl.BlockSpec((H, D), lambda i: (0, 0)),
            pl.BlockSpec((1, D), lambda i: (0, 0)),
        ],
        out_specs=pl.BlockSpec((TILE, D), lambda i: (i, 0)),
        out_shape=jax.ShapeDtypeStruct((T, D), jnp.float32),
    )(xf, Wg, bg.reshape(1, E), W1, b1.reshape(1, H), W2, b2.reshape(1, D))
    return out.reshape(B, N, D)
